# baseline (speedup 1.0000x reference)
import functools

import jax
import jax.numpy as jnp
from jax import lax
from jax.experimental import pallas as pl
from jax.experimental.pallas import tpu as pltpu
from jax.experimental.pallas import tpu_sc as plsc

N_NODES = 10000
N_EDGES = 160000
NC = 2
NS = 16
NW = NC * NS
EPW = N_EDGES // NW
CH = 125
NCHUNK = EPW // CH
STRIDE = 624
ZROWS = 128
NBUF = 8

_MESH = plsc.VectorSubcoreMesh(
    core_axis_name="c", subcore_axis_name="s", num_cores=NC, num_subcores=NS
)


def _make_spmm(d_feat: int):
    zero_stores = (ZROWS * d_feat) // 16

    @functools.partial(
        pl.kernel,
        out_type=jax.ShapeDtypeStruct((N_NODES, NC * d_feat), jnp.float32),
        mesh=_MESH,
        compiler_params=pltpu.CompilerParams(use_tc_tiling_on_sc=False),
        scratch_types=[
            pltpu.VMEM((NCHUNK, CH), jnp.int32),
            pltpu.VMEM((NCHUNK, CH), jnp.int32),
            pltpu.VMEM((NBUF, CH, d_feat), jnp.float32),
            pltpu.VMEM((ZROWS, d_feat), jnp.float32),
            pltpu.VMEM_SHARED((N_NODES, d_feat), jnp.float32),
            pltpu.SemaphoreType.DMA((NBUF,)),
            pltpu.SemaphoreType.DMA((NBUF,)),
        ],
    )
    def spmm(h_hbm, src_hbm, dst_hbm, out_hbm,
             sidx, didx, rows, zbuf, acc, gsem, ssem):
        c = lax.axis_index("c")
        s = lax.axis_index("s")
        wid = c * NS + s

        pltpu.sync_copy(src_hbm.at[wid], sidx)
        pltpu.sync_copy(dst_hbm.at[wid], didx)

        zeros16 = jnp.zeros((16,), jnp.float32)

        def zbody(k, _):
            i = k // (d_feat // 16)
            j = (k % (d_feat // 16)) * 16
            zbuf[i, pl.ds(j, 16)] = zeros16
            return 0

        lax.fori_loop(0, zero_stores, zbody, 0)
        for k in range(5):
            pltpu.sync_copy(zbuf, acc.at[pl.ds(s * STRIDE + k * ZROWS, ZROWS)])
        plsc.subcore_barrier()

        def body(i, _):
            gathers = []
            for k in range(NBUF):
                @pl.when(i > 0)
                def _():
                    pltpu.make_async_copy(
                        rows.at[k], acc.at[didx.at[0]], ssem.at[k]
                    ).wait()

                gathers.append(
                    pltpu.async_copy(
                        h_hbm.at[sidx.at[NBUF * i + k]], rows.at[k], gsem.at[k]
                    )
                )
            for k in range(NBUF):
                gathers[k].wait()
                pltpu.async_copy(
                    rows.at[k], acc.at[didx.at[NBUF * i + k]], ssem.at[k],
                    add=True,
                )
            return 0

        lax.fori_loop(0, NCHUNK // NBUF, body, 0)
        for k in range(NBUF):
            pltpu.make_async_copy(
                rows.at[k], acc.at[didx.at[0]], ssem.at[k]
            ).wait()
        plsc.subcore_barrier()

        pltpu.sync_copy(
            acc.at[pl.ds(s * STRIDE, STRIDE)],
            out_hbm.at[pl.ds(s * STRIDE, STRIDE), pl.ds(c * d_feat, d_feat)],
        )

        @pl.when(s == NS - 1)
        def _():
            pltpu.sync_copy(
                acc.at[pl.ds(NS * STRIDE, N_NODES - NS * STRIDE)],
                out_hbm.at[pl.ds(NS * STRIDE, N_NODES - NS * STRIDE),
                           pl.ds(c * d_feat, d_feat)],
            )

    return spmm


_spmm64 = _make_spmm(64)
_spmm16 = _make_spmm(16)


_ROWS_BLK = 1000
_GRID = N_NODES // _ROWS_BLK


def _dense1_body(x_ref, w1_ref, W1_ref, u_ref):
    u_ref[...] = jnp.dot(
        x_ref[...] * w1_ref[...],
        W1_ref[...],
        preferred_element_type=jnp.float32,
        precision=lax.Precision.HIGHEST,
    )


def _dense1(x, w1, W1):
    return pl.pallas_call(
        _dense1_body,
        grid=(_GRID,),
        in_specs=[
            pl.BlockSpec((_ROWS_BLK, 256), lambda i: (i, 0)),
            pl.BlockSpec((1, 256), lambda i: (0, 0)),
            pl.BlockSpec((256, 64), lambda i: (0, 0)),
        ],
        out_specs=pl.BlockSpec((_ROWS_BLK, 64), lambda i: (i, 0)),
        out_shape=jax.ShapeDtypeStruct((N_NODES, 64), jnp.float32),
    )(x, w1, W1)


def _dense2_body(v_ref, M_ref, b1w2_ref, W2_ref, Wo16_ref, t_ref):
    m = jnp.dot(v_ref[...], M_ref[...], preferred_element_type=jnp.float32,
                precision=lax.Precision.HIGHEST)
    m = m + jnp.dot(b1w2_ref[...], W2_ref[...],
                    preferred_element_type=jnp.float32,
                    precision=lax.Precision.HIGHEST)
    t_ref[...] = jnp.dot(m, Wo16_ref[...], preferred_element_type=jnp.float32,
                         precision=lax.Precision.HIGHEST)


def _dense2(vcat, M, b1w2, W2, Wo16):
    return pl.pallas_call(
        _dense2_body,
        grid=(_GRID,),
        in_specs=[
            pl.BlockSpec((_ROWS_BLK, 128), lambda i: (i, 0)),
            pl.BlockSpec((128, 32), lambda i: (0, 0)),
            pl.BlockSpec((1, 64), lambda i: (0, 0)),
            pl.BlockSpec((64, 32), lambda i: (0, 0)),
            pl.BlockSpec((32, 16), lambda i: (0, 0)),
        ],
        out_specs=pl.BlockSpec((_ROWS_BLK, 16), lambda i: (i, 0)),
        out_shape=jax.ShapeDtypeStruct((N_NODES, 16), jnp.float32),
    )(vcat, M, b1w2, W2, Wo16)


def _softmax_body(r_ref, b2r_ref, Wo16_ref, bo16_ref, out_ref):
    cvec = jnp.dot(b2r_ref[...], Wo16_ref[...],
                   preferred_element_type=jnp.float32,
                   precision=lax.Precision.HIGHEST) + bo16_ref[...]
    logits = r_ref[:, :16] + r_ref[:, 16:32] + cvec
    col = lax.broadcasted_iota(jnp.int32, logits.shape, 1)
    logits = jnp.where(col < 4, logits, -1e30)
    m = jnp.max(logits, axis=1, keepdims=True)
    e = jnp.exp(logits - m)
    p = e / jnp.sum(e, axis=1, keepdims=True)
    out_ref[...] = p[:, :4]


def _softmax(rcat, b2r, Wo16, bo16):
    return pl.pallas_call(
        _softmax_body,
        grid=(_GRID,),
        in_specs=[
            pl.BlockSpec((_ROWS_BLK, 32), lambda i: (i, 0)),
            pl.BlockSpec((1, 32), lambda i: (0, 0)),
            pl.BlockSpec((32, 16), lambda i: (0, 0)),
            pl.BlockSpec((1, 16), lambda i: (0, 0)),
        ],
        out_specs=pl.BlockSpec((_ROWS_BLK, 4), lambda i: (i, 0)),
        out_shape=jax.ShapeDtypeStruct((N_NODES, 4), jnp.float32),
    )(rcat, b2r, Wo16, bo16)


def kernel(x, edge_index, w1, W1, b1, w2, W2, b2, Wo, bo):
    src = edge_index[1].astype(jnp.int32).reshape(NW, NCHUNK, CH)
    dst = edge_index[0].astype(jnp.int32).reshape(NW, NCHUNK, CH)
    Wo16 = jnp.concatenate([Wo, jnp.zeros((32, 12), jnp.float32)], axis=1)
    bo16 = jnp.concatenate([bo, jnp.zeros((12,), jnp.float32)]).reshape(1, 16)
    dW2 = w2.reshape(64, 1) * W2
    M = jnp.concatenate([dW2, dW2], axis=0)
    b1w2 = (b1 * w2).reshape(1, 64)
    b2r = b2.reshape(1, 32)

    u = _dense1(x, w1, W1)
    vcat = _spmm64(u, src, dst)
    t = _dense2(vcat, M, b1w2, W2, Wo16)
    rcat = _spmm16(t, src, dst)
    return _softmax(rcat, b2r, Wo16, bo16)


# speedup vs baseline: 1.1367x; 1.1367x over previous
import functools

import jax
import jax.numpy as jnp
from jax import lax
from jax.experimental import pallas as pl
from jax.experimental.pallas import tpu as pltpu
from jax.experimental.pallas import tpu_sc as plsc

N_NODES = 10000
N_EDGES = 160000
NC = 2
NS = 16
NW = NC * NS
EPW = N_EDGES // NW
CH = 125
NCHUNK = EPW // CH
STRIDE = 624
ZROWS = 128
NBUF = 8

_MESH = plsc.VectorSubcoreMesh(
    core_axis_name="c", subcore_axis_name="s", num_cores=NC, num_subcores=NS
)


def _make_spmm(d_feat: int):
    zero_stores = (ZROWS * d_feat) // 16

    @functools.partial(
        pl.kernel,
        out_type=jax.ShapeDtypeStruct((N_NODES, NC * d_feat), jnp.float32),
        mesh=_MESH,
        compiler_params=pltpu.CompilerParams(use_tc_tiling_on_sc=False),
        scratch_types=[
            pltpu.VMEM((NCHUNK, CH), jnp.int32),
            pltpu.VMEM((NCHUNK, CH), jnp.int32),
            pltpu.VMEM((NBUF, CH, d_feat), jnp.float32),
            pltpu.VMEM((ZROWS, d_feat), jnp.float32),
            pltpu.VMEM_SHARED((N_NODES, d_feat), jnp.float32),
            pltpu.SemaphoreType.DMA((NBUF,)),
            pltpu.SemaphoreType.DMA((NBUF,)),
        ],
    )
    def spmm(h_hbm, src_hbm, dst_hbm, out_hbm,
             sidx, didx, rows, zbuf, acc, gsem, ssem):
        c = lax.axis_index("c")
        s = lax.axis_index("s")
        wid = c * NS + s

        pltpu.sync_copy(src_hbm.at[wid], sidx)
        pltpu.sync_copy(dst_hbm.at[wid], didx)

        zeros16 = jnp.zeros((16,), jnp.float32)

        def zbody(k, _):
            i = k // (d_feat // 16)
            j = (k % (d_feat // 16)) * 16
            zbuf[i, pl.ds(j, 16)] = zeros16
            return 0

        lax.fori_loop(0, zero_stores, zbody, 0)
        for k in range(5):
            pltpu.sync_copy(zbuf, acc.at[pl.ds(s * STRIDE + k * ZROWS, ZROWS)])
        plsc.subcore_barrier()

        def body(i, _):
            gathers = []
            for k in range(NBUF):
                @pl.when(i > 0)
                def _():
                    pltpu.make_async_copy(
                        rows.at[k], acc.at[didx.at[0]], ssem.at[k]
                    ).wait()

                gathers.append(
                    pltpu.async_copy(
                        h_hbm.at[sidx.at[NBUF * i + k]], rows.at[k], gsem.at[k]
                    )
                )
            for k in range(NBUF):
                gathers[k].wait()
                pltpu.async_copy(
                    rows.at[k], acc.at[didx.at[NBUF * i + k]], ssem.at[k],
                    add=True,
                )
            return 0

        lax.fori_loop(0, NCHUNK // NBUF, body, 0)
        for k in range(NBUF):
            pltpu.make_async_copy(
                rows.at[k], acc.at[didx.at[0]], ssem.at[k]
            ).wait()
        plsc.subcore_barrier()

        pltpu.sync_copy(
            acc.at[pl.ds(s * STRIDE, STRIDE)],
            out_hbm.at[pl.ds(s * STRIDE, STRIDE), pl.ds(c * d_feat, d_feat)],
        )

        @pl.when(s == NS - 1)
        def _():
            pltpu.sync_copy(
                acc.at[pl.ds(NS * STRIDE, N_NODES - NS * STRIDE)],
                out_hbm.at[pl.ds(NS * STRIDE, N_NODES - NS * STRIDE),
                           pl.ds(c * d_feat, d_feat)],
            )

    return spmm


_spmm64 = _make_spmm(64)
_spmm16 = _make_spmm(16)


_ROWS_BLK = 2000
_GRID = N_NODES // _ROWS_BLK


def _dense1_body(x_ref, w1_ref, W1_ref, u_ref):
    u_ref[...] = jnp.dot(
        x_ref[...] * w1_ref[...],
        W1_ref[...],
        preferred_element_type=jnp.float32,
        precision=lax.Precision.HIGHEST,
    )


def _dense1(x, w1, W1):
    return pl.pallas_call(
        _dense1_body,
        grid=(_GRID,),
        in_specs=[
            pl.BlockSpec((_ROWS_BLK, 256), lambda i: (i, 0)),
            pl.BlockSpec((1, 256), lambda i: (0, 0)),
            pl.BlockSpec((256, 64), lambda i: (0, 0)),
        ],
        out_specs=pl.BlockSpec((_ROWS_BLK, 64), lambda i: (i, 0)),
        out_shape=jax.ShapeDtypeStruct((N_NODES, 64), jnp.float32),
    )(x, w1, W1)


def _dense2_body(v_ref, M_ref, b1w2_ref, W2_ref, Wo16_ref, t_ref):
    m = jnp.dot(v_ref[...], M_ref[...], preferred_element_type=jnp.float32,
                precision=lax.Precision.HIGHEST)
    m = m + jnp.dot(b1w2_ref[...], W2_ref[...],
                    preferred_element_type=jnp.float32,
                    precision=lax.Precision.HIGHEST)
    t_ref[...] = jnp.dot(m, Wo16_ref[...], preferred_element_type=jnp.float32,
                         precision=lax.Precision.HIGHEST)


def _dense2(vcat, M, b1w2, W2, Wo16):
    return pl.pallas_call(
        _dense2_body,
        grid=(_GRID,),
        in_specs=[
            pl.BlockSpec((_ROWS_BLK, 128), lambda i: (i, 0)),
            pl.BlockSpec((128, 32), lambda i: (0, 0)),
            pl.BlockSpec((1, 64), lambda i: (0, 0)),
            pl.BlockSpec((64, 32), lambda i: (0, 0)),
            pl.BlockSpec((32, 16), lambda i: (0, 0)),
        ],
        out_specs=pl.BlockSpec((_ROWS_BLK, 16), lambda i: (i, 0)),
        out_shape=jax.ShapeDtypeStruct((N_NODES, 16), jnp.float32),
    )(vcat, M, b1w2, W2, Wo16)


def _softmax_body(r_ref, b2r_ref, Wo16_ref, bo16_ref, out_ref):
    cvec = jnp.dot(b2r_ref[...], Wo16_ref[...],
                   preferred_element_type=jnp.float32,
                   precision=lax.Precision.HIGHEST) + bo16_ref[...]
    logits = r_ref[:, :16] + r_ref[:, 16:32] + cvec
    col = lax.broadcasted_iota(jnp.int32, logits.shape, 1)
    logits = jnp.where(col < 4, logits, -1e30)
    m = jnp.max(logits, axis=1, keepdims=True)
    e = jnp.exp(logits - m)
    p = e / jnp.sum(e, axis=1, keepdims=True)
    out_ref[...] = p[:, :4]


def _softmax(rcat, b2r, Wo16, bo16):
    return pl.pallas_call(
        _softmax_body,
        grid=(_GRID,),
        in_specs=[
            pl.BlockSpec((_ROWS_BLK, 32), lambda i: (i, 0)),
            pl.BlockSpec((1, 32), lambda i: (0, 0)),
            pl.BlockSpec((32, 16), lambda i: (0, 0)),
            pl.BlockSpec((1, 16), lambda i: (0, 0)),
        ],
        out_specs=pl.BlockSpec((_ROWS_BLK, 4), lambda i: (i, 0)),
        out_shape=jax.ShapeDtypeStruct((N_NODES, 4), jnp.float32),
    )(rcat, b2r, Wo16, bo16)


def kernel(x, edge_index, w1, W1, b1, w2, W2, b2, Wo, bo):
    src = edge_index[1].astype(jnp.int32).reshape(NW, NCHUNK, CH)
    dst = edge_index[0].astype(jnp.int32).reshape(NW, NCHUNK, CH)
    Wo16 = jnp.concatenate([Wo, jnp.zeros((32, 12), jnp.float32)], axis=1)
    bo16 = jnp.concatenate([bo, jnp.zeros((12,), jnp.float32)]).reshape(1, 16)
    dW2 = w2.reshape(64, 1) * W2
    M = jnp.concatenate([dW2, dW2], axis=0)
    b1w2 = (b1 * w2).reshape(1, 64)
    b2r = b2.reshape(1, 32)

    u = _dense1(x, w1, W1)
    vcat = _spmm64(u, src, dst)
    t = _dense2(vcat, M, b1w2, W2, Wo16)
    rcat = _spmm16(t, src, dst)
    return _softmax(rcat, b2r, Wo16, bo16)


# speedup vs baseline: 1.1378x; 1.0010x over previous
import functools

import jax
import jax.numpy as jnp
from jax import lax
from jax.experimental import pallas as pl
from jax.experimental.pallas import tpu as pltpu
from jax.experimental.pallas import tpu_sc as plsc

N_NODES = 10000
N_EDGES = 160000
NC = 2
NS = 16
NW = NC * NS
EPW = N_EDGES // NW
CH = 125
NCHUNK = EPW // CH
STRIDE = 624
ZROWS = 128
_NBUF_BY_D = {64: 8, 16: 8}

_MESH = plsc.VectorSubcoreMesh(
    core_axis_name="c", subcore_axis_name="s", num_cores=NC, num_subcores=NS
)


def _make_spmm(d_feat: int):
    zero_stores = (ZROWS * d_feat) // 16
    NBUF = _NBUF_BY_D[d_feat]

    @functools.partial(
        pl.kernel,
        out_type=jax.ShapeDtypeStruct((N_NODES, NC * d_feat), jnp.float32),
        mesh=_MESH,
        compiler_params=pltpu.CompilerParams(use_tc_tiling_on_sc=False),
        scratch_types=[
            pltpu.VMEM((NCHUNK, CH), jnp.int32),
            pltpu.VMEM((NCHUNK, CH), jnp.int32),
            pltpu.VMEM((NBUF, CH, d_feat), jnp.float32),
            pltpu.VMEM((ZROWS, d_feat), jnp.float32),
            pltpu.VMEM_SHARED((N_NODES, d_feat), jnp.float32),
            pltpu.SemaphoreType.DMA((NBUF,)),
            pltpu.SemaphoreType.DMA((NBUF,)),
        ],
    )
    def spmm(h_hbm, src_hbm, dst_hbm, out_hbm,
             sidx, didx, rows, zbuf, acc, gsem, ssem):
        c = lax.axis_index("c")
        s = lax.axis_index("s")
        wid = c * NS + s

        pltpu.sync_copy(src_hbm.at[wid], sidx)
        pltpu.sync_copy(dst_hbm.at[wid], didx)

        zeros16 = jnp.zeros((16,), jnp.float32)

        def zbody(k, _):
            i = k // (d_feat // 16)
            j = (k % (d_feat // 16)) * 16
            zbuf[i, pl.ds(j, 16)] = zeros16
            return 0

        lax.fori_loop(0, zero_stores, zbody, 0)
        for k in range(5):
            pltpu.sync_copy(zbuf, acc.at[pl.ds(s * STRIDE + k * ZROWS, ZROWS)])
        plsc.subcore_barrier()

        def body(i, _):
            gathers = []
            for k in range(NBUF):
                @pl.when(i > 0)
                def _():
                    pltpu.make_async_copy(
                        rows.at[k], acc.at[didx.at[0]], ssem.at[k]
                    ).wait()

                gathers.append(
                    pltpu.async_copy(
                        h_hbm.at[sidx.at[NBUF * i + k]], rows.at[k], gsem.at[k]
                    )
                )
            for k in range(NBUF):
                gathers[k].wait()
                pltpu.async_copy(
                    rows.at[k], acc.at[didx.at[NBUF * i + k]], ssem.at[k],
                    add=True,
                )
            return 0

        lax.fori_loop(0, NCHUNK // NBUF, body, 0)
        for k in range(NBUF):
            pltpu.make_async_copy(
                rows.at[k], acc.at[didx.at[0]], ssem.at[k]
            ).wait()
        plsc.subcore_barrier()

        pltpu.sync_copy(
            acc.at[pl.ds(s * STRIDE, STRIDE)],
            out_hbm.at[pl.ds(s * STRIDE, STRIDE), pl.ds(c * d_feat, d_feat)],
        )

        @pl.when(s == NS - 1)
        def _():
            pltpu.sync_copy(
                acc.at[pl.ds(NS * STRIDE, N_NODES - NS * STRIDE)],
                out_hbm.at[pl.ds(NS * STRIDE, N_NODES - NS * STRIDE),
                           pl.ds(c * d_feat, d_feat)],
            )

    return spmm


_spmm64 = _make_spmm(64)
_spmm16 = _make_spmm(16)


_ROWS_BLK = 2000
_GRID = N_NODES // _ROWS_BLK


def _dense1_body(x_ref, w1_ref, W1_ref, u_ref):
    u_ref[...] = jnp.dot(
        x_ref[...] * w1_ref[...],
        W1_ref[...],
        preferred_element_type=jnp.float32,
        precision=lax.Precision.HIGHEST,
    )


def _dense1(x, w1, W1):
    return pl.pallas_call(
        _dense1_body,
        grid=(_GRID,),
        in_specs=[
            pl.BlockSpec((_ROWS_BLK, 256), lambda i: (i, 0)),
            pl.BlockSpec((1, 256), lambda i: (0, 0)),
            pl.BlockSpec((256, 64), lambda i: (0, 0)),
        ],
        out_specs=pl.BlockSpec((_ROWS_BLK, 64), lambda i: (i, 0)),
        out_shape=jax.ShapeDtypeStruct((N_NODES, 64), jnp.float32),
    )(x, w1, W1)


def _dense2_body(v_ref, M_ref, b1w2_ref, W2_ref, Wo16_ref, t_ref):
    m = jnp.dot(v_ref[...], M_ref[...], preferred_element_type=jnp.float32,
                precision=lax.Precision.HIGHEST)
    m = m + jnp.dot(b1w2_ref[...], W2_ref[...],
                    preferred_element_type=jnp.float32,
                    precision=lax.Precision.HIGHEST)
    t_ref[...] = jnp.dot(m, Wo16_ref[...], preferred_element_type=jnp.float32,
                         precision=lax.Precision.HIGHEST)


def _dense2(vcat, M, b1w2, W2, Wo16):
    return pl.pallas_call(
        _dense2_body,
        grid=(_GRID,),
        in_specs=[
            pl.BlockSpec((_ROWS_BLK, 128), lambda i: (i, 0)),
            pl.BlockSpec((128, 32), lambda i: (0, 0)),
            pl.BlockSpec((1, 64), lambda i: (0, 0)),
            pl.BlockSpec((64, 32), lambda i: (0, 0)),
            pl.BlockSpec((32, 16), lambda i: (0, 0)),
        ],
        out_specs=pl.BlockSpec((_ROWS_BLK, 16), lambda i: (i, 0)),
        out_shape=jax.ShapeDtypeStruct((N_NODES, 16), jnp.float32),
    )(vcat, M, b1w2, W2, Wo16)


def _softmax_body(r_ref, b2r_ref, Wo16_ref, bo16_ref, out_ref):
    cvec = jnp.dot(b2r_ref[...], Wo16_ref[...],
                   preferred_element_type=jnp.float32,
                   precision=lax.Precision.HIGHEST) + bo16_ref[...]
    logits = r_ref[:, :16] + r_ref[:, 16:32] + cvec
    col = lax.broadcasted_iota(jnp.int32, logits.shape, 1)
    logits = jnp.where(col < 4, logits, -1e30)
    m = jnp.max(logits, axis=1, keepdims=True)
    e = jnp.exp(logits - m)
    p = e / jnp.sum(e, axis=1, keepdims=True)
    out_ref[...] = p[:, :4]


def _softmax(rcat, b2r, Wo16, bo16):
    return pl.pallas_call(
        _softmax_body,
        grid=(_GRID,),
        in_specs=[
            pl.BlockSpec((_ROWS_BLK, 32), lambda i: (i, 0)),
            pl.BlockSpec((1, 32), lambda i: (0, 0)),
            pl.BlockSpec((32, 16), lambda i: (0, 0)),
            pl.BlockSpec((1, 16), lambda i: (0, 0)),
        ],
        out_specs=pl.BlockSpec((_ROWS_BLK, 4), lambda i: (i, 0)),
        out_shape=jax.ShapeDtypeStruct((N_NODES, 4), jnp.float32),
    )(rcat, b2r, Wo16, bo16)


def kernel(x, edge_index, w1, W1, b1, w2, W2, b2, Wo, bo):
    src = edge_index[1].astype(jnp.int32).reshape(NW, NCHUNK, CH)
    dst = edge_index[0].astype(jnp.int32).reshape(NW, NCHUNK, CH)
    Wo16 = jnp.concatenate([Wo, jnp.zeros((32, 12), jnp.float32)], axis=1)
    bo16 = jnp.concatenate([bo, jnp.zeros((12,), jnp.float32)]).reshape(1, 16)
    dW2 = w2.reshape(64, 1) * W2
    M = jnp.concatenate([dW2, dW2], axis=0)
    b1w2 = (b1 * w2).reshape(1, 64)
    b2r = b2.reshape(1, 32)

    u = _dense1(x, w1, W1)
    vcat = _spmm64(u, src, dst)
    t = _dense2(vcat, M, b1w2, W2, Wo16)
    rcat = _spmm16(t, src, dst)
    return _softmax(rcat, b2r, Wo16, bo16)


# speedup vs baseline: 1.3217x; 1.1616x over previous
import functools

import jax
import jax.numpy as jnp
from jax import lax
from jax.experimental import pallas as pl
from jax.experimental.pallas import tpu as pltpu
from jax.experimental.pallas import tpu_sc as plsc

N_NODES = 10000
N_EDGES = 160000
NC = 2
NS = 16
NW = NC * NS
EPW = N_EDGES // NW
CH = 125
NCHUNK = EPW // CH
STRIDE = 624
ZROWS = 128
_NBUF_BY_D = {64: 8, 16: 8}

_MESH = plsc.VectorSubcoreMesh(
    core_axis_name="c", subcore_axis_name="s", num_cores=NC, num_subcores=NS
)


def _make_spmm(d_feat: int):
    zero_stores = (ZROWS * d_feat) // 16
    NBUF = _NBUF_BY_D[d_feat]

    @functools.partial(
        pl.kernel,
        out_type=jax.ShapeDtypeStruct((N_NODES, NC * d_feat), jnp.float32),
        mesh=_MESH,
        compiler_params=pltpu.CompilerParams(use_tc_tiling_on_sc=False),
        scratch_types=[
            pltpu.VMEM((NCHUNK, CH), jnp.int32),
            pltpu.VMEM((NCHUNK, CH), jnp.int32),
            pltpu.VMEM((NBUF, CH, d_feat), jnp.float32),
            pltpu.VMEM((ZROWS, d_feat), jnp.float32),
            pltpu.VMEM_SHARED((N_NODES, d_feat), jnp.float32),
            pltpu.SemaphoreType.DMA((NBUF,)),
            pltpu.SemaphoreType.DMA((NBUF,)),
        ],
    )
    def spmm(h_hbm, src_hbm, dst_hbm, out_hbm,
             sidx, didx, rows, zbuf, acc, gsem, ssem):
        c = lax.axis_index("c")
        s = lax.axis_index("s")
        wid = c * NS + s

        pltpu.sync_copy(src_hbm.at[wid], sidx)
        pltpu.sync_copy(dst_hbm.at[wid], didx)

        for k in range(NBUF):
            pltpu.async_copy(h_hbm.at[sidx.at[k]], rows.at[k], gsem.at[k])

        zeros16 = jnp.zeros((16,), jnp.float32)

        def zbody(k, _):
            i = k // (d_feat // 16)
            j = (k % (d_feat // 16)) * 16
            zbuf[i, pl.ds(j, 16)] = zeros16
            return 0

        lax.fori_loop(0, zero_stores, zbody, 0)
        for k in range(5):
            pltpu.sync_copy(zbuf, acc.at[pl.ds(s * STRIDE + k * ZROWS, ZROWS)])
        plsc.subcore_barrier()

        def body(i, _):
            for k in range(NBUF):
                @pl.when(i > 0)
                def _():
                    pltpu.make_async_copy(
                        rows.at[k], acc.at[didx.at[0]], ssem.at[k]
                    ).wait()
                    pltpu.async_copy(
                        h_hbm.at[sidx.at[NBUF * i + k]], rows.at[k], gsem.at[k]
                    )
            for k in range(NBUF):
                pltpu.make_async_copy(
                    h_hbm.at[sidx.at[0]], rows.at[k], gsem.at[k]
                ).wait()
                pltpu.async_copy(
                    rows.at[k], acc.at[didx.at[NBUF * i + k]], ssem.at[k],
                    add=True,
                )
            return 0

        lax.fori_loop(0, NCHUNK // NBUF, body, 0)
        for k in range(NBUF):
            pltpu.make_async_copy(
                rows.at[k], acc.at[didx.at[0]], ssem.at[k]
            ).wait()
        plsc.subcore_barrier()

        pltpu.sync_copy(
            acc.at[pl.ds(s * STRIDE, STRIDE)],
            out_hbm.at[pl.ds(s * STRIDE, STRIDE), pl.ds(c * d_feat, d_feat)],
        )

        @pl.when(s == NS - 1)
        def _():
            pltpu.sync_copy(
                acc.at[pl.ds(NS * STRIDE, N_NODES - NS * STRIDE)],
                out_hbm.at[pl.ds(NS * STRIDE, N_NODES - NS * STRIDE),
                           pl.ds(c * d_feat, d_feat)],
            )

    return spmm


_spmm64 = _make_spmm(64)
_spmm16 = _make_spmm(16)


_ROWS_BLK = 2000
_GRID = N_NODES // _ROWS_BLK


def _dense1_body(x_ref, w1_ref, W1_ref, u_ref):
    u_ref[...] = jnp.dot(
        x_ref[...] * w1_ref[...],
        W1_ref[...],
        preferred_element_type=jnp.float32,
    )


def _dense1(x, w1, W1):
    return pl.pallas_call(
        _dense1_body,
        grid=(_GRID,),
        in_specs=[
            pl.BlockSpec((_ROWS_BLK, 256), lambda i: (i, 0)),
            pl.BlockSpec((1, 256), lambda i: (0, 0)),
            pl.BlockSpec((256, 64), lambda i: (0, 0)),
        ],
        out_specs=pl.BlockSpec((_ROWS_BLK, 64), lambda i: (i, 0)),
        out_shape=jax.ShapeDtypeStruct((N_NODES, 64), jnp.float32),
    )(x, w1, W1)


def _dense2_body(v_ref, M_ref, b1w2_ref, W2_ref, Wo16_ref, t_ref):
    m = jnp.dot(v_ref[...], M_ref[...], preferred_element_type=jnp.float32)
    m = m + jnp.dot(b1w2_ref[...], W2_ref[...],
                    preferred_element_type=jnp.float32)
    t_ref[...] = jnp.dot(m, Wo16_ref[...], preferred_element_type=jnp.float32)


def _dense2(vcat, M, b1w2, W2, Wo16):
    return pl.pallas_call(
        _dense2_body,
        grid=(_GRID,),
        in_specs=[
            pl.BlockSpec((_ROWS_BLK, 128), lambda i: (i, 0)),
            pl.BlockSpec((128, 32), lambda i: (0, 0)),
            pl.BlockSpec((1, 64), lambda i: (0, 0)),
            pl.BlockSpec((64, 32), lambda i: (0, 0)),
            pl.BlockSpec((32, 16), lambda i: (0, 0)),
        ],
        out_specs=pl.BlockSpec((_ROWS_BLK, 16), lambda i: (i, 0)),
        out_shape=jax.ShapeDtypeStruct((N_NODES, 16), jnp.float32),
    )(vcat, M, b1w2, W2, Wo16)


def _softmax_body(r_ref, b2r_ref, Wo16_ref, bo16_ref, out_ref):
    cvec = jnp.dot(b2r_ref[...], Wo16_ref[...],
                   preferred_element_type=jnp.float32,
                   ) + bo16_ref[...]
    logits = r_ref[:, :16] + r_ref[:, 16:32] + cvec
    col = lax.broadcasted_iota(jnp.int32, logits.shape, 1)
    logits = jnp.where(col < 4, logits, -1e30)
    m = jnp.max(logits, axis=1, keepdims=True)
    e = jnp.exp(logits - m)
    p = e / jnp.sum(e, axis=1, keepdims=True)
    out_ref[...] = p[:, :4]


def _softmax(rcat, b2r, Wo16, bo16):
    return pl.pallas_call(
        _softmax_body,
        grid=(_GRID,),
        in_specs=[
            pl.BlockSpec((_ROWS_BLK, 32), lambda i: (i, 0)),
            pl.BlockSpec((1, 32), lambda i: (0, 0)),
            pl.BlockSpec((32, 16), lambda i: (0, 0)),
            pl.BlockSpec((1, 16), lambda i: (0, 0)),
        ],
        out_specs=pl.BlockSpec((_ROWS_BLK, 4), lambda i: (i, 0)),
        out_shape=jax.ShapeDtypeStruct((N_NODES, 4), jnp.float32),
    )(rcat, b2r, Wo16, bo16)


def kernel(x, edge_index, w1, W1, b1, w2, W2, b2, Wo, bo):
    src = edge_index[1].astype(jnp.int32).reshape(NW, NCHUNK, CH)
    dst = edge_index[0].astype(jnp.int32).reshape(NW, NCHUNK, CH)
    Wo16 = jnp.concatenate([Wo, jnp.zeros((32, 12), jnp.float32)], axis=1)
    bo16 = jnp.concatenate([bo, jnp.zeros((12,), jnp.float32)]).reshape(1, 16)
    dW2 = w2.reshape(64, 1) * W2
    M = jnp.concatenate([dW2, dW2], axis=0)
    b1w2 = (b1 * w2).reshape(1, 64)
    b2r = b2.reshape(1, 32)

    u = _dense1(x, w1, W1)
    vcat = _spmm64(u, src, dst)
    t = _dense2(vcat, M, b1w2, W2, Wo16)
    rcat = _spmm16(t, src, dst)
    return _softmax(rcat, b2r, Wo16, bo16)
